# initial kernel scaffold (unmeasured)
import functools

import jax
import jax.numpy as jnp
from jax import lax
from jax.experimental import pallas as pl
from jax.experimental.pallas import tpu as pltpu

N_DEV = 4
B_PER = 2
SQ = 512
SKV = 512
HQ = 32
H_PER = HQ // N_DEV
DH = 64
DM = 768
F_PER = H_PER * DH
WINDOW = 128
SCALE = 0.125


def kernel(x, Wq, K_ext, V_ext, Wo):
    my = lax.axis_index("i")

    k_s = lax.dynamic_slice_in_dim(K_ext, my * B_PER, B_PER, axis=0)
    v_s = lax.dynamic_slice_in_dim(V_ext, my * B_PER, B_PER, axis=0)
    kt = k_s.transpose(2, 0, 1, 3).reshape(N_DEV, H_PER, B_PER, SKV, DH)
    vt = v_s.transpose(2, 0, 1, 3).reshape(N_DEV, H_PER, B_PER, SKV, DH)
    order = (my - jnp.arange(N_DEV)) % N_DEV
    k_proc = kt[order]
    v_proc = vt[order]

    def body(x_ref, wq_ref, k_ref, v_ref, wo_ref, out_ref,
             wq_buf, wo_buf, wq_ssem, wq_rsem, wo_ssem, wo_rsem):
        my_pos = lax.axis_index("i")
        left = (my_pos - 1) % N_DEV
        right = (my_pos + 1) % N_DEV

        barrier = pltpu.get_barrier_semaphore()
        for nbr in (left, right):
            pl.semaphore_signal(barrier, inc=1, device_id=(nbr,),
                                device_id_type=pl.DeviceIdType.MESH)
        pl.semaphore_wait(barrier, 2)

        qi = lax.broadcasted_iota(jnp.int32, (SQ, SKV), 0)
        ki = lax.broadcasted_iota(jnp.int32, (SQ, SKV), 1)
        mask = jnp.abs(qi - ki) <= WINDOW

        def compute_chunk(h, wq_c, wo_c):
            for b in range(B_PER):
                q = jnp.dot(x_ref[b], wq_c, preferred_element_type=jnp.float32)
                ctxs = []
                for hh in range(H_PER):
                    qh = q[:, hh * DH:(hh + 1) * DH]
                    s = lax.dot_general(
                        qh, k_ref[h, hh, b], (((1,), (1,)), ((), ())),
                        preferred_element_type=jnp.float32)
                    s = jnp.where(mask, s * SCALE, -1e9)
                    m = jnp.max(s, axis=-1, keepdims=True)
                    w = jnp.exp(s - m)
                    w = w / jnp.sum(w, axis=-1, keepdims=True)
                    ctxs.append(jnp.dot(w, v_ref[h, hh, b],
                                        preferred_element_type=jnp.float32))
                ctx = jnp.concatenate(ctxs, axis=1)
                contrib = jnp.dot(ctx, wo_c, preferred_element_type=jnp.float32)
                if h == 0:
                    out_ref[b] = contrib
                else:
                    out_ref[b] = out_ref[b] + contrib

        for h in range(N_DEV):
            if h < N_DEV - 1:
                src_q = wq_ref if h == 0 else wq_buf.at[h - 1]
                src_o = wo_ref if h == 0 else wo_buf.at[h - 1]
                r_q = pltpu.make_async_remote_copy(
                    src_ref=src_q, dst_ref=wq_buf.at[h],
                    send_sem=wq_ssem.at[h], recv_sem=wq_rsem.at[h],
                    device_id=(right,), device_id_type=pl.DeviceIdType.MESH)
                r_o = pltpu.make_async_remote_copy(
                    src_ref=src_o, dst_ref=wo_buf.at[h],
                    send_sem=wo_ssem.at[h], recv_sem=wo_rsem.at[h],
                    device_id=(right,), device_id_type=pl.DeviceIdType.MESH)
                r_q.start()
                r_o.start()
            if h == 0:
                compute_chunk(h, wq_ref[...], wo_ref[...])
            else:
                compute_chunk(h, wq_buf[h - 1], wo_buf[h - 1])
            if h < N_DEV - 1:
                r_q.wait()
                r_o.wait()

        @functools.partial(pl.run_scoped, sem=pltpu.SemaphoreType.REGULAR)
        def _(sem):
            for nbr in (left, right):
                pl.semaphore_signal(sem, inc=1, device_id=(nbr,),
                                    device_id_type=pl.DeviceIdType.MESH)
            pl.semaphore_wait(sem, 2)

    return pl.pallas_call(
        body,
        out_shape=jax.ShapeDtypeStruct((B_PER, SQ, DM), jnp.float32),
        in_specs=[pl.BlockSpec(memory_space=pltpu.VMEM)] * 5,
        out_specs=pl.BlockSpec(memory_space=pltpu.VMEM),
        scratch_shapes=[
            pltpu.VMEM((N_DEV - 1, DM, F_PER), jnp.float32),
            pltpu.VMEM((N_DEV - 1, F_PER, DM), jnp.float32),
            pltpu.SemaphoreType.DMA((N_DEV - 1,)),
            pltpu.SemaphoreType.DMA((N_DEV - 1,)),
            pltpu.SemaphoreType.DMA((N_DEV - 1,)),
            pltpu.SemaphoreType.DMA((N_DEV - 1,)),
        ],
        compiler_params=pltpu.CompilerParams(collective_id=0),
    )(x, Wq, k_proc, v_proc, Wo)


# baseline (device time: 172584 ns/iter reference)
import functools

import jax
import jax.numpy as jnp
from jax import lax
from jax.experimental import pallas as pl
from jax.experimental.pallas import tpu as pltpu

N_DEV = 4
B_PER = 2
SQ = 512
SKV = 512
HQ = 32
H_PER = HQ // N_DEV
DH = 64
DM = 768
F_PER = H_PER * DH
WINDOW = 128
SCALE = 0.125


def kernel(x, Wq, K_ext, V_ext, Wo):
    my = lax.axis_index("i")

    k_s = lax.dynamic_slice_in_dim(K_ext, my * B_PER, B_PER, axis=0)
    v_s = lax.dynamic_slice_in_dim(V_ext, my * B_PER, B_PER, axis=0)
    kt = k_s.transpose(2, 0, 1, 3).reshape(N_DEV, H_PER, B_PER, SKV, DH)
    vt = v_s.transpose(2, 0, 1, 3).reshape(N_DEV, H_PER, B_PER, SKV, DH)
    order = (my - jnp.arange(N_DEV)) % N_DEV
    k_proc = kt[order]
    v_proc = vt[order]

    def body(x_ref, wq_ref, k_ref, v_ref, wo_ref, out_ref,
             wq_buf, wo_buf, wq_ssem, wq_rsem, wo_ssem, wo_rsem):
        my_pos = lax.axis_index("i")
        left = (my_pos - 1) % N_DEV
        right = (my_pos + 1) % N_DEV

        barrier = pltpu.get_barrier_semaphore()
        for nbr in (left, right):
            pl.semaphore_signal(barrier, inc=1, device_id=(nbr,),
                                device_id_type=pl.DeviceIdType.MESH)
        pl.semaphore_wait(barrier, 2)

        qi = lax.broadcasted_iota(jnp.int32, (SQ, SKV), 0)
        ki = lax.broadcasted_iota(jnp.int32, (SQ, SKV), 1)
        mask = jnp.abs(qi - ki) <= WINDOW

        def compute_chunk(h, wq_c, wo_c):
            for b in range(B_PER):
                q = jnp.dot(x_ref[b], wq_c, preferred_element_type=jnp.float32)
                ctxs = []
                for hh in range(H_PER):
                    qh = q[:, hh * DH:(hh + 1) * DH]
                    s = lax.dot_general(
                        qh, k_ref[h, hh, b], (((1,), (1,)), ((), ())),
                        preferred_element_type=jnp.float32)
                    s = jnp.where(mask, s * SCALE, -1e9)
                    m = jnp.max(s, axis=-1, keepdims=True)
                    w = jnp.exp(s - m)
                    w = w / jnp.sum(w, axis=-1, keepdims=True)
                    ctxs.append(jnp.dot(w, v_ref[h, hh, b],
                                        preferred_element_type=jnp.float32))
                ctx = jnp.concatenate(ctxs, axis=1)
                contrib = jnp.dot(ctx, wo_c, preferred_element_type=jnp.float32)
                if h == 0:
                    out_ref[b] = contrib
                else:
                    out_ref[b] = out_ref[b] + contrib

        for h in range(N_DEV):
            if h < N_DEV - 1:
                src_q = wq_ref if h == 0 else wq_buf.at[h - 1]
                src_o = wo_ref if h == 0 else wo_buf.at[h - 1]
                r_q = pltpu.make_async_remote_copy(
                    src_ref=src_q, dst_ref=wq_buf.at[h],
                    send_sem=wq_ssem.at[h], recv_sem=wq_rsem.at[h],
                    device_id=(right,), device_id_type=pl.DeviceIdType.MESH)
                r_o = pltpu.make_async_remote_copy(
                    src_ref=src_o, dst_ref=wo_buf.at[h],
                    send_sem=wo_ssem.at[h], recv_sem=wo_rsem.at[h],
                    device_id=(right,), device_id_type=pl.DeviceIdType.MESH)
                r_q.start()
                r_o.start()
            if h == 0:
                compute_chunk(h, wq_ref[...], wo_ref[...])
            else:
                compute_chunk(h, wq_buf[h - 1], wo_buf[h - 1])
            if h < N_DEV - 1:
                r_q.wait()
                r_o.wait()

        @functools.partial(pl.run_scoped, sem=pltpu.SemaphoreType.REGULAR)
        def _(sem):
            for nbr in (left, right):
                pl.semaphore_signal(sem, inc=1, device_id=(nbr,),
                                    device_id_type=pl.DeviceIdType.MESH)
            pl.semaphore_wait(sem, 2)

    return pl.pallas_call(
        body,
        out_shape=jax.ShapeDtypeStruct((B_PER, SQ, DM), jnp.float32),
        in_specs=[pl.BlockSpec(memory_space=pltpu.VMEM)] * 5,
        out_specs=pl.BlockSpec(memory_space=pltpu.VMEM),
        scratch_shapes=[
            pltpu.VMEM((N_DEV - 1, DM, F_PER), jnp.float32),
            pltpu.VMEM((N_DEV - 1, F_PER, DM), jnp.float32),
            pltpu.SemaphoreType.DMA((N_DEV - 1,)),
            pltpu.SemaphoreType.DMA((N_DEV - 1,)),
            pltpu.SemaphoreType.DMA((N_DEV - 1,)),
            pltpu.SemaphoreType.DMA((N_DEV - 1,)),
        ],
        compiler_params=pltpu.CompilerParams(
            collective_id=0, vmem_limit_bytes=100 * 1024 * 1024),
    )(x, Wq, k_proc, v_proc, Wo)


# device time: 104039 ns/iter; 1.6588x vs baseline; 1.6588x over previous
import functools

import jax
import jax.numpy as jnp
from jax import lax
from jax.experimental import pallas as pl
from jax.experimental.pallas import tpu as pltpu

N_DEV = 4
B_PER = 2
SQ = 512
SKV = 512
HQ = 32
H_PER = HQ // N_DEV
DH = 64
DM = 768
F_PER = H_PER * DH
WINDOW = 128
SCALE = 0.125


def kernel(x, Wq, K_ext, V_ext, Wo):
    my = lax.axis_index("i")

    k_s = lax.dynamic_slice_in_dim(K_ext, my * B_PER, B_PER, axis=0)
    v_s = lax.dynamic_slice_in_dim(V_ext, my * B_PER, B_PER, axis=0)
    kt = k_s.transpose(2, 0, 1, 3).reshape(N_DEV, H_PER, B_PER, SKV, DH)
    vt = v_s.transpose(2, 0, 1, 3).reshape(N_DEV, H_PER, B_PER, SKV, DH)
    order = (my - jnp.arange(N_DEV)) % N_DEV
    k_proc = kt[order].astype(jnp.bfloat16)
    v_proc = vt[order].astype(jnp.bfloat16)
    x_bf = x.astype(jnp.bfloat16)
    wq_bf = Wq.astype(jnp.bfloat16)
    wo_bf = Wo.astype(jnp.bfloat16)

    def body(x_ref, wq_ref, k_ref, v_ref, wo_ref, out_ref,
             wq_buf, wo_buf, wq_ssem, wq_rsem, wo_ssem, wo_rsem):
        my_pos = lax.axis_index("i")
        left = (my_pos - 1) % N_DEV
        right = (my_pos + 1) % N_DEV

        barrier = pltpu.get_barrier_semaphore()
        for nbr in (left, right):
            pl.semaphore_signal(barrier, inc=1, device_id=(nbr,),
                                device_id_type=pl.DeviceIdType.MESH)
        pl.semaphore_wait(barrier, 2)

        qi = lax.broadcasted_iota(jnp.int32, (SQ, SKV), 0)
        ki = lax.broadcasted_iota(jnp.int32, (SQ, SKV), 1)
        maskf = (jnp.abs(qi - ki) <= WINDOW).astype(jnp.float32)

        def compute_chunk(h, wq_c, wo_c):
            for b in range(B_PER):
                q = jnp.dot(x_ref[b], wq_c,
                            preferred_element_type=jnp.float32
                            ).astype(jnp.bfloat16)
                ctxs = []
                for hh in range(H_PER):
                    qh = q[:, hh * DH:(hh + 1) * DH]
                    s = lax.dot_general(
                        qh, k_ref[h, hh, b], (((1,), (1,)), ((), ())),
                        preferred_element_type=jnp.float32)
                    w = jnp.exp(s * SCALE) * maskf
                    r = 1.0 / jnp.sum(w, axis=-1, keepdims=True)
                    ctx_h = jnp.dot(w.astype(jnp.bfloat16), v_ref[h, hh, b],
                                    preferred_element_type=jnp.float32)
                    ctxs.append((ctx_h * r).astype(jnp.bfloat16))
                ctx = jnp.concatenate(ctxs, axis=1)
                contrib = jnp.dot(ctx, wo_c, preferred_element_type=jnp.float32)
                if h == 0:
                    out_ref[b] = contrib
                else:
                    out_ref[b] = out_ref[b] + contrib

        for h in range(N_DEV):
            if h < N_DEV - 1:
                src_q = wq_ref if h == 0 else wq_buf.at[h - 1]
                src_o = wo_ref if h == 0 else wo_buf.at[h - 1]
                r_q = pltpu.make_async_remote_copy(
                    src_ref=src_q, dst_ref=wq_buf.at[h],
                    send_sem=wq_ssem.at[h], recv_sem=wq_rsem.at[h],
                    device_id=(right,), device_id_type=pl.DeviceIdType.MESH)
                r_o = pltpu.make_async_remote_copy(
                    src_ref=src_o, dst_ref=wo_buf.at[h],
                    send_sem=wo_ssem.at[h], recv_sem=wo_rsem.at[h],
                    device_id=(right,), device_id_type=pl.DeviceIdType.MESH)
                r_q.start()
                r_o.start()
            if h == 0:
                compute_chunk(h, wq_ref[...], wo_ref[...])
            else:
                compute_chunk(h, wq_buf[h - 1], wo_buf[h - 1])
            if h < N_DEV - 1:
                r_q.wait()
                r_o.wait()

        @functools.partial(pl.run_scoped, sem=pltpu.SemaphoreType.REGULAR)
        def _(sem):
            for nbr in (left, right):
                pl.semaphore_signal(sem, inc=1, device_id=(nbr,),
                                    device_id_type=pl.DeviceIdType.MESH)
            pl.semaphore_wait(sem, 2)

    return pl.pallas_call(
        body,
        out_shape=jax.ShapeDtypeStruct((B_PER, SQ, DM), jnp.float32),
        in_specs=[pl.BlockSpec(memory_space=pltpu.VMEM)] * 5,
        out_specs=pl.BlockSpec(memory_space=pltpu.VMEM),
        scratch_shapes=[
            pltpu.VMEM((N_DEV - 1, DM, F_PER), jnp.bfloat16),
            pltpu.VMEM((N_DEV - 1, F_PER, DM), jnp.bfloat16),
            pltpu.SemaphoreType.DMA((N_DEV - 1,)),
            pltpu.SemaphoreType.DMA((N_DEV - 1,)),
            pltpu.SemaphoreType.DMA((N_DEV - 1,)),
            pltpu.SemaphoreType.DMA((N_DEV - 1,)),
        ],
        compiler_params=pltpu.CompilerParams(
            collective_id=0, vmem_limit_bytes=100 * 1024 * 1024),
    )(x_bf, wq_bf, k_proc, v_proc, wo_bf)


# device time: 77481 ns/iter; 2.2274x vs baseline; 1.3428x over previous
import functools

import jax
import jax.numpy as jnp
from jax import lax
from jax.experimental import pallas as pl
from jax.experimental.pallas import tpu as pltpu

N_DEV = 4
B_PER = 2
SQ = 512
SKV = 512
HQ = 32
H_PER = HQ // N_DEV
DH = 64
DM = 768
F_PER = H_PER * DH
WINDOW = 128
SCALE = 0.125
WQ_HALF = DM // 2
WO_HALF = F_PER // 2


def kernel(x, Wq, K_ext, V_ext, Wo):
    my = lax.axis_index("i")

    k_s = lax.dynamic_slice_in_dim(K_ext, my * B_PER, B_PER, axis=0)
    v_s = lax.dynamic_slice_in_dim(V_ext, my * B_PER, B_PER, axis=0)
    k_bf = (k_s.transpose(2, 0, 1, 3)
            .reshape(N_DEV, H_PER, B_PER, SKV, DH).astype(jnp.bfloat16))
    v_bf = (v_s.transpose(2, 0, 1, 3)
            .reshape(N_DEV, H_PER, B_PER, SKV, DH).astype(jnp.bfloat16))
    x_bf = x.astype(jnp.bfloat16)
    wq_bf = Wq.astype(jnp.bfloat16)
    wo_bf = Wo.astype(jnp.bfloat16)

    def body(x_ref, wq_ref, k_ref, v_ref, wo_ref, out_ref,
             wq_bufL, wq_bufR, wq_bufO, wo_bufL, wo_bufR, wo_bufO,
             p1_ssem, p1_rsem, p2_ssem, p2_rsem):
        my_pos = lax.axis_index("i")
        left = (my_pos - 1) % N_DEV
        right = (my_pos + 1) % N_DEV
        opp = (my_pos + 2) % N_DEV

        barrier = pltpu.get_barrier_semaphore()
        for nbr in (left, right):
            pl.semaphore_signal(barrier, inc=1, device_id=(nbr,),
                                device_id_type=pl.DeviceIdType.MESH)
        pl.semaphore_wait(barrier, 2)

        qi = lax.broadcasted_iota(jnp.int32, (SQ, SKV), 0)
        ki = lax.broadcasted_iota(jnp.int32, (SQ, SKV), 1)
        maskf = (jnp.abs(qi - ki) <= WINDOW).astype(jnp.float32)

        def compute_chunk(c, wq_c, wo_c, first):
            for b in range(B_PER):
                q = jnp.dot(x_ref[b], wq_c,
                            preferred_element_type=jnp.float32
                            ).astype(jnp.bfloat16)
                ctxs = []
                for hh in range(H_PER):
                    qh = q[:, hh * DH:(hh + 1) * DH]
                    s = lax.dot_general(
                        qh, k_ref[c, hh, b], (((1,), (1,)), ((), ())),
                        preferred_element_type=jnp.float32)
                    w = jnp.exp(s * SCALE) * maskf
                    r = 1.0 / jnp.sum(w, axis=-1, keepdims=True)
                    ctx_h = jnp.dot(w.astype(jnp.bfloat16), v_ref[c, hh, b],
                                    preferred_element_type=jnp.float32)
                    ctxs.append((ctx_h * r).astype(jnp.bfloat16))
                ctx = jnp.concatenate(ctxs, axis=1)
                contrib = jnp.dot(ctx, wo_c, preferred_element_type=jnp.float32)
                if first:
                    out_ref[b] = contrib
                else:
                    out_ref[b] = out_ref[b] + contrib

        p1 = []
        for idx, (src, dst, tgt) in enumerate([
            (wq_ref, wq_bufL, right),
            (wq_ref, wq_bufR, left),
            (wo_ref, wo_bufL, right),
            (wo_ref, wo_bufR, left),
        ]):
            r = pltpu.make_async_remote_copy(
                src_ref=src, dst_ref=dst,
                send_sem=p1_ssem.at[idx], recv_sem=p1_rsem.at[idx],
                device_id=(tgt,), device_id_type=pl.DeviceIdType.MESH)
            r.start()
            p1.append(r)

        compute_chunk(my_pos, wq_ref[...], wo_ref[...], first=True)

        for r in p1:
            r.wait_recv()

        p2 = []
        for idx, (src, dst, tgt) in enumerate([
            (wq_bufR.at[pl.ds(0, WQ_HALF)], wq_bufO.at[pl.ds(0, WQ_HALF)], left),
            (wq_bufL.at[pl.ds(WQ_HALF, WQ_HALF)],
             wq_bufO.at[pl.ds(WQ_HALF, WQ_HALF)], right),
            (wo_bufR.at[pl.ds(0, WO_HALF)], wo_bufO.at[pl.ds(0, WO_HALF)], left),
            (wo_bufL.at[pl.ds(WO_HALF, WO_HALF)],
             wo_bufO.at[pl.ds(WO_HALF, WO_HALF)], right),
        ]):
            r = pltpu.make_async_remote_copy(
                src_ref=src, dst_ref=dst,
                send_sem=p2_ssem.at[idx], recv_sem=p2_rsem.at[idx],
                device_id=(tgt,), device_id_type=pl.DeviceIdType.MESH)
            r.start()
            p2.append(r)

        compute_chunk(left, wq_bufL[...], wo_bufL[...], first=False)
        compute_chunk(right, wq_bufR[...], wo_bufR[...], first=False)

        for r in p2:
            r.wait_recv()

        compute_chunk(opp, wq_bufO[...], wo_bufO[...], first=False)

        for r in p1 + p2:
            r.wait_send()

        @functools.partial(pl.run_scoped, sem=pltpu.SemaphoreType.REGULAR)
        def _(sem):
            for nbr in (left, right):
                pl.semaphore_signal(sem, inc=1, device_id=(nbr,),
                                    device_id_type=pl.DeviceIdType.MESH)
            pl.semaphore_wait(sem, 2)

    return pl.pallas_call(
        body,
        out_shape=jax.ShapeDtypeStruct((B_PER, SQ, DM), jnp.float32),
        in_specs=[pl.BlockSpec(memory_space=pltpu.VMEM)] * 5,
        out_specs=pl.BlockSpec(memory_space=pltpu.VMEM),
        scratch_shapes=[
            pltpu.VMEM((DM, F_PER), jnp.bfloat16),
            pltpu.VMEM((DM, F_PER), jnp.bfloat16),
            pltpu.VMEM((DM, F_PER), jnp.bfloat16),
            pltpu.VMEM((F_PER, DM), jnp.bfloat16),
            pltpu.VMEM((F_PER, DM), jnp.bfloat16),
            pltpu.VMEM((F_PER, DM), jnp.bfloat16),
            pltpu.SemaphoreType.DMA((4,)),
            pltpu.SemaphoreType.DMA((4,)),
            pltpu.SemaphoreType.DMA((4,)),
            pltpu.SemaphoreType.DMA((4,)),
        ],
        compiler_params=pltpu.CompilerParams(
            collective_id=0, vmem_limit_bytes=100 * 1024 * 1024),
    )(x_bf, wq_bf, k_bf, v_bf, wo_bf)
